# baseline (speedup 1.0000x reference)
import jax
import jax.numpy as jnp
from jax import lax
from jax.experimental import pallas as pl
from jax.experimental.pallas import tpu as pltpu
from jax.experimental.pallas import tpu_sc as plsc
import functools

N = 100_000
E = 6_400_000
LANE = 128
ROWS = E // LANE
NC, NS = 2, 16
NW = NC * NS
RPW = 1560
XTRA = (ROWS - RPW * NW) // 8
CH = 32
NCHUNK = 49
N_PAD = 102_400
SLC = N_PAD // NS


def _sc_body(eidx_hbm, w_hbm, x_hbm, out_hbm,
             x_v, src_v0, dst_v0, w_v0, src_v1, dst_v1, w_v1, acc_sh,
             sem0, sem1, lsem0, lsem1):
    cid = lax.axis_index("c")
    sid = lax.axis_index("s")
    wid = sid * NC + cid

    def _zero(i, c):
        x_v[pl.ds(i * 16, 16)] = jnp.zeros((16,), jnp.float32)
        return c
    lax.fori_loop(0, SLC // 16, _zero, 0)
    off = sid * SLC
    pltpu.sync_copy(x_v.at[pl.ds(0, SLC)], acc_sh.at[pl.ds(off, SLC)])

    pltpu.sync_copy(x_hbm, x_v)
    plsc.subcore_barrier()

    my_rows = RPW + 8 * jnp.where(wid < XTRA, 1, 0)
    r0 = wid * RPW + 8 * jnp.minimum(wid, XTRA)

    def _gather_row(j, src_v, w_v):
        for k in range(LANE // 16):
            sl = pl.ds(k * 16, 16)
            idx = src_v[j, sl]
            vals = plsc.load_gather(x_v, [idx])
            w_v[j, sl] = w_v[j, sl] * vals

    def _gather_fire_rows(lo, hi, src_v, dst_v, w_v, sem, full=True):
        if full:
            def _row(j, cc):
                _gather_row(j, src_v, w_v)
                return cc
            lax.fori_loop(0, CH, _row, 0, unroll=8)
            for j in range(CH):
                pltpu.async_copy(w_v.at[j], acc_sh.at[dst_v.at[j]], sem,
                                 add=True)
        else:
            def _rowf(j, cc):
                _gather_row(j, src_v, w_v)
                pltpu.async_copy(w_v.at[j], acc_sh.at[dst_v.at[j]], sem,
                                 add=True)
                return cc
            lax.fori_loop(lo, hi, _rowf, 0)

    def _issue_loads(base, src_v, dst_v, w_v, lsem):
        pltpu.async_copy(eidx_hbm.at[0, pl.ds(base, CH), :], src_v, lsem)
        pltpu.async_copy(eidx_hbm.at[1, pl.ds(base, CH), :], dst_v, lsem)
        pltpu.async_copy(w_hbm.at[pl.ds(base, CH), :], w_v, lsem)

    def _wait_loads(src_v, dst_v, w_v, lsem):
        pltpu.make_async_copy(eidx_hbm.at[0, pl.ds(0, CH), :], src_v, lsem).wait()
        pltpu.make_async_copy(eidx_hbm.at[1, pl.ds(0, CH), :], dst_v, lsem).wait()
        pltpu.make_async_copy(w_hbm.at[pl.ds(0, CH), :], w_v, lsem).wait()

    def _drain(dst_v, w_v, sem):
        pltpu.make_async_copy(w_hbm.at[pl.ds(0, CH), :], w_v, sem).wait()

    set0 = (src_v0, dst_v0, w_v0)
    set1 = (src_v1, dst_v1, w_v1)

    _issue_loads(pl.multiple_of(r0, 8), *set0, lsem0)

    def _pair(t, carry):
        _wait_loads(*set0, lsem0)
        _gather_fire_rows(0, CH, *set0, sem0)
        pl.when(t > 0)(lambda: _drain(dst_v1, w_v1, sem1))
        _issue_loads(pl.multiple_of(r0 + (2 * t + 1) * CH, 8), *set1, lsem1)
        _wait_loads(*set1, lsem1)
        _gather_fire_rows(0, CH, *set1, sem1)
        _drain(dst_v0, w_v0, sem0)
        nb = jnp.minimum(r0 + (2 * t + 2) * CH, r0 + my_rows - CH)
        _issue_loads(pl.multiple_of(nb, 8), *set0, lsem0)
        return carry
    lax.fori_loop(0, (NCHUNK - 1) // 2, _pair, 0)

    skip = NCHUNK * CH - my_rows
    _wait_loads(*set0, lsem0)

    def _zfire(j, cc):
        for k in range(LANE // 16):
            w_v0[j, pl.ds(k * 16, 16)] = jnp.zeros((16,), jnp.float32)
        pltpu.async_copy(w_v0.at[j], acc_sh.at[dst_v0.at[j]], sem0, add=True)
        return cc
    lax.fori_loop(0, skip, _zfire, 0)
    _gather_fire_rows(skip, CH, *set0, sem0, full=False)
    _drain(dst_v1, w_v1, sem1)
    _drain(dst_v0, w_v0, sem0)

    plsc.subcore_barrier()
    obase = pl.multiple_of(cid * N_PAD + off, 8)
    pltpu.sync_copy(acc_sh.at[pl.ds(off, SLC)], out_hbm.at[pl.ds(obase, SLC)])


@functools.partial(jax.jit, static_argnames=())
def _sc_edges(eidx, w, x):
    mesh = plsc.VectorSubcoreMesh(core_axis_name="c", subcore_axis_name="s",
                                  num_cores=NC, num_subcores=NS)
    return pl.kernel(
        _sc_body,
        out_type=jax.ShapeDtypeStruct((NC * N_PAD,), jnp.float32),
        mesh=mesh,
        compiler_params=pltpu.CompilerParams(needs_layout_passes=False),
        scratch_types=[
            pltpu.VMEM((N,), jnp.float32),
            pltpu.VMEM((CH, LANE), jnp.int32),
            pltpu.VMEM((CH, LANE), jnp.int32),
            pltpu.VMEM((CH, LANE), jnp.float32),
            pltpu.VMEM((CH, LANE), jnp.int32),
            pltpu.VMEM((CH, LANE), jnp.int32),
            pltpu.VMEM((CH, LANE), jnp.float32),
            pltpu.VMEM_SHARED((N_PAD,), jnp.float32),
            pltpu.SemaphoreType.DMA,
            pltpu.SemaphoreType.DMA,
            pltpu.SemaphoreType.DMA,
            pltpu.SemaphoreType.DMA,
        ],
    )(eidx, w, x)


def _tc_body(x_ref, p_ref, o_ref):
    o_ref[...] = jnp.tanh(x_ref[...] + p_ref[0] + p_ref[1])


def _tc_combine(xp, p):
    return pl.pallas_call(
        _tc_body,
        out_shape=jax.ShapeDtypeStruct((N_PAD // LANE, LANE), jnp.float32),
    )(xp, p)


def kernel(region_inputs_flat, edge_index, edge_weight):
    x = region_inputs_flat
    eidx = edge_index.astype(jnp.int32).reshape(2, ROWS, LANE)
    w = edge_weight.reshape(ROWS, LANE)
    partial = _sc_edges(eidx, w, x)
    xp = jnp.concatenate([x, jnp.zeros((N_PAD - N,), jnp.float32)])
    out2 = _tc_combine(xp.reshape(N_PAD // LANE, LANE),
                       partial.reshape(NC, N_PAD // LANE, LANE))
    return out2.reshape(-1)[:N]


# speedup vs baseline: 1.1084x; 1.1084x over previous
import jax
import jax.numpy as jnp
from jax import lax
from jax.experimental import pallas as pl
from jax.experimental.pallas import tpu as pltpu
from jax.experimental.pallas import tpu_sc as plsc
import functools

N = 100_000
E = 6_400_000
LANE = 128
ROWS = E // LANE
NC, NS = 2, 16
NW = NC * NS
RPW = 1560
XTRA = (ROWS - RPW * NW) // 8
CH = 32
NCHUNK = 49
N_PAD = 102_400
SLC = N_PAD // NS


def _sc_body(eidx_hbm, w_hbm, x_hbm, out_hbm,
             x_v, src_v0, dst_v0, w_v0, src_v1, dst_v1, w_v1, acc_sh,
             sem0, sem1, lsem0, lsem1):
    cid = lax.axis_index("c")
    sid = lax.axis_index("s")
    wid = sid * NC + cid

    def _zero(i, c):
        x_v[pl.ds(i * 16, 16)] = jnp.zeros((16,), jnp.float32)
        return c
    lax.fori_loop(0, SLC // 16, _zero, 0)
    off = sid * SLC
    pltpu.sync_copy(x_v.at[pl.ds(0, SLC)], acc_sh.at[pl.ds(off, SLC)])

    pltpu.sync_copy(x_hbm, x_v)
    plsc.subcore_barrier()

    my_rows = RPW + 8 * jnp.where(wid < XTRA, 1, 0)
    r0 = wid * RPW + 8 * jnp.minimum(wid, XTRA)

    def _gather_row(j, src_v, w_v):
        for k in range(LANE // 16):
            sl = pl.ds(k * 16, 16)
            idx = src_v[j, sl]
            vals = plsc.load_gather(x_v, [idx])
            w_v[j, sl] = w_v[j, sl] * vals

    def _gather_fire_rows(lo, hi, src_v, dst_v, w_v, sem, full=True):
        if full:
            def _row(j, cc):
                _gather_row(j, src_v, w_v)
                pltpu.async_copy(w_v.at[j], acc_sh.at[dst_v.at[j]], sem,
                                 add=True)
                return cc
            lax.fori_loop(0, CH, _row, 0, unroll=2)
        else:
            def _rowf(j, cc):
                _gather_row(j, src_v, w_v)
                pltpu.async_copy(w_v.at[j], acc_sh.at[dst_v.at[j]], sem,
                                 add=True)
                return cc
            lax.fori_loop(lo, hi, _rowf, 0)

    def _issue_loads(base, src_v, dst_v, w_v, lsem):
        pltpu.async_copy(eidx_hbm.at[0, pl.ds(base, CH), :], src_v, lsem)
        pltpu.async_copy(eidx_hbm.at[1, pl.ds(base, CH), :], dst_v, lsem)
        pltpu.async_copy(w_hbm.at[pl.ds(base, CH), :], w_v, lsem)

    def _wait_loads(src_v, dst_v, w_v, lsem):
        pltpu.make_async_copy(eidx_hbm.at[0, pl.ds(0, CH), :], src_v, lsem).wait()
        pltpu.make_async_copy(eidx_hbm.at[1, pl.ds(0, CH), :], dst_v, lsem).wait()
        pltpu.make_async_copy(w_hbm.at[pl.ds(0, CH), :], w_v, lsem).wait()

    def _drain(dst_v, w_v, sem):
        pltpu.make_async_copy(w_hbm.at[pl.ds(0, CH), :], w_v, sem).wait()

    set0 = (src_v0, dst_v0, w_v0)
    set1 = (src_v1, dst_v1, w_v1)

    _issue_loads(pl.multiple_of(r0, 8), *set0, lsem0)

    def _pair(t, carry):
        _wait_loads(*set0, lsem0)
        _gather_fire_rows(0, CH, *set0, sem0)
        pl.when(t > 0)(lambda: _drain(dst_v1, w_v1, sem1))
        _issue_loads(pl.multiple_of(r0 + (2 * t + 1) * CH, 8), *set1, lsem1)
        _wait_loads(*set1, lsem1)
        _gather_fire_rows(0, CH, *set1, sem1)
        _drain(dst_v0, w_v0, sem0)
        nb = jnp.minimum(r0 + (2 * t + 2) * CH, r0 + my_rows - CH)
        _issue_loads(pl.multiple_of(nb, 8), *set0, lsem0)
        return carry
    lax.fori_loop(0, (NCHUNK - 1) // 2, _pair, 0)

    skip = NCHUNK * CH - my_rows
    _wait_loads(*set0, lsem0)

    def _zfire(j, cc):
        for k in range(LANE // 16):
            w_v0[j, pl.ds(k * 16, 16)] = jnp.zeros((16,), jnp.float32)
        pltpu.async_copy(w_v0.at[j], acc_sh.at[dst_v0.at[j]], sem0, add=True)
        return cc
    lax.fori_loop(0, skip, _zfire, 0)
    _gather_fire_rows(skip, CH, *set0, sem0, full=False)
    _drain(dst_v1, w_v1, sem1)
    _drain(dst_v0, w_v0, sem0)

    plsc.subcore_barrier()
    obase = pl.multiple_of(cid * N_PAD + off, 8)
    pltpu.sync_copy(acc_sh.at[pl.ds(off, SLC)], out_hbm.at[pl.ds(obase, SLC)])


@functools.partial(jax.jit, static_argnames=())
def _sc_edges(eidx, w, x):
    mesh = plsc.VectorSubcoreMesh(core_axis_name="c", subcore_axis_name="s",
                                  num_cores=NC, num_subcores=NS)
    return pl.kernel(
        _sc_body,
        out_type=jax.ShapeDtypeStruct((NC * N_PAD,), jnp.float32),
        mesh=mesh,
        compiler_params=pltpu.CompilerParams(needs_layout_passes=False),
        scratch_types=[
            pltpu.VMEM((N,), jnp.float32),
            pltpu.VMEM((CH, LANE), jnp.int32),
            pltpu.VMEM((CH, LANE), jnp.int32),
            pltpu.VMEM((CH, LANE), jnp.float32),
            pltpu.VMEM((CH, LANE), jnp.int32),
            pltpu.VMEM((CH, LANE), jnp.int32),
            pltpu.VMEM((CH, LANE), jnp.float32),
            pltpu.VMEM_SHARED((N_PAD,), jnp.float32),
            pltpu.SemaphoreType.DMA,
            pltpu.SemaphoreType.DMA,
            pltpu.SemaphoreType.DMA,
            pltpu.SemaphoreType.DMA,
        ],
    )(eidx, w, x)


def _tc_body(x_ref, p_ref, o_ref):
    o_ref[...] = jnp.tanh(x_ref[...] + p_ref[0] + p_ref[1])


def _tc_combine(xp, p):
    return pl.pallas_call(
        _tc_body,
        out_shape=jax.ShapeDtypeStruct((N_PAD // LANE, LANE), jnp.float32),
    )(xp, p)


def kernel(region_inputs_flat, edge_index, edge_weight):
    x = region_inputs_flat
    eidx = edge_index.astype(jnp.int32).reshape(2, ROWS, LANE)
    w = edge_weight.reshape(ROWS, LANE)
    partial = _sc_edges(eidx, w, x)
    xp = jnp.concatenate([x, jnp.zeros((N_PAD - N,), jnp.float32)])
    out2 = _tc_combine(xp.reshape(N_PAD // LANE, LANE),
                       partial.reshape(NC, N_PAD // LANE, LANE))
    return out2.reshape(-1)[:N]


# speedup vs baseline: 1.7207x; 1.5524x over previous
import jax
import jax.numpy as jnp
from jax import lax
from jax.experimental import pallas as pl
from jax.experimental.pallas import tpu as pltpu
from jax.experimental.pallas import tpu_sc as plsc
import functools

N = 100_000
E = 6_400_000
LANE = 128
ROWS = E // LANE
NC, NS = 2, 16
NW = NC * NS
RPW = 1560
XTRA = (ROWS - RPW * NW) // 8
CH = 32
NCHUNK = 49
N_PAD = 102_400
SLC = N_PAD // NS


def _sc_body(eidx_hbm, w_hbm, x_hbm, out_hbm,
             x_v, src_v0, dst_v0, w_v0, src_v1, dst_v1, w_v1, acc_sh,
             sem0, sem1, lsem0, lsem1):
    cid = lax.axis_index("c")
    sid = lax.axis_index("s")
    wid = sid * NC + cid

    def _zero(i, c):
        x_v[pl.ds(i * 16, 16)] = jnp.zeros((16,), jnp.float32)
        return c
    lax.fori_loop(0, SLC // 16, _zero, 0)
    off = sid * SLC
    pltpu.sync_copy(x_v.at[pl.ds(0, SLC)], acc_sh.at[pl.ds(off, SLC)])

    pltpu.sync_copy(x_hbm, x_v)
    plsc.subcore_barrier()

    my_rows = RPW + 8 * jnp.where(wid < XTRA, 1, 0)
    r0 = wid * RPW + 8 * jnp.minimum(wid, XTRA)

    def _gather_row(j, src_v, w_v):
        sls = [pl.ds(k * 16, 16) for k in range(LANE // 16)]
        idxs = [src_v[j, sl] for sl in sls]
        vals = [plsc.load_gather(x_v, [idx]) for idx in idxs]
        for sl, v in zip(sls, vals):
            w_v[j, sl] = w_v[j, sl] * v

    def _gather_fire_rows(lo, hi, src_v, dst_v, w_v, sem, full=True):
        if full:
            def _row(j, cc):
                _gather_row(j, src_v, w_v)
                pltpu.async_copy(w_v.at[j], acc_sh.at[dst_v.at[j]], sem,
                                 add=True)
                return cc
            lax.fori_loop(0, CH, _row, 0)
        else:
            def _rowf(j, cc):
                _gather_row(j, src_v, w_v)
                pltpu.async_copy(w_v.at[j], acc_sh.at[dst_v.at[j]], sem,
                                 add=True)
                return cc
            lax.fori_loop(lo, hi, _rowf, 0)

    def _issue_loads(base, src_v, dst_v, w_v, lsem):
        pltpu.async_copy(eidx_hbm.at[0, pl.ds(base, CH), :], src_v, lsem)
        pltpu.async_copy(eidx_hbm.at[1, pl.ds(base, CH), :], dst_v, lsem)
        pltpu.async_copy(w_hbm.at[pl.ds(base, CH), :], w_v, lsem)

    def _wait_loads(src_v, dst_v, w_v, lsem):
        pltpu.make_async_copy(eidx_hbm.at[0, pl.ds(0, CH), :], src_v, lsem).wait()
        pltpu.make_async_copy(eidx_hbm.at[1, pl.ds(0, CH), :], dst_v, lsem).wait()
        pltpu.make_async_copy(w_hbm.at[pl.ds(0, CH), :], w_v, lsem).wait()

    def _drain(dst_v, w_v, sem):
        pltpu.make_async_copy(w_hbm.at[pl.ds(0, CH), :], w_v, sem).wait()

    set0 = (src_v0, dst_v0, w_v0)
    set1 = (src_v1, dst_v1, w_v1)

    _issue_loads(pl.multiple_of(r0, 8), *set0, lsem0)

    def _pair(t, carry):
        _wait_loads(*set0, lsem0)
        _gather_fire_rows(0, CH, *set0, sem0)
        pl.when(t > 0)(lambda: _drain(dst_v1, w_v1, sem1))
        _issue_loads(pl.multiple_of(r0 + (2 * t + 1) * CH, 8), *set1, lsem1)
        _wait_loads(*set1, lsem1)
        _gather_fire_rows(0, CH, *set1, sem1)
        _drain(dst_v0, w_v0, sem0)
        nb = jnp.minimum(r0 + (2 * t + 2) * CH, r0 + my_rows - CH)
        _issue_loads(pl.multiple_of(nb, 8), *set0, lsem0)
        return carry
    lax.fori_loop(0, (NCHUNK - 1) // 2, _pair, 0)

    skip = NCHUNK * CH - my_rows
    _wait_loads(*set0, lsem0)

    def _zfire(j, cc):
        for k in range(LANE // 16):
            w_v0[j, pl.ds(k * 16, 16)] = jnp.zeros((16,), jnp.float32)
        pltpu.async_copy(w_v0.at[j], acc_sh.at[dst_v0.at[j]], sem0, add=True)
        return cc
    lax.fori_loop(0, skip, _zfire, 0)
    _gather_fire_rows(skip, CH, *set0, sem0, full=False)
    _drain(dst_v1, w_v1, sem1)
    _drain(dst_v0, w_v0, sem0)

    plsc.subcore_barrier()
    obase = pl.multiple_of(cid * N_PAD + off, 8)
    pltpu.sync_copy(acc_sh.at[pl.ds(off, SLC)], out_hbm.at[pl.ds(obase, SLC)])


@functools.partial(jax.jit, static_argnames=())
def _sc_edges(eidx, w, x):
    mesh = plsc.VectorSubcoreMesh(core_axis_name="c", subcore_axis_name="s",
                                  num_cores=NC, num_subcores=NS)
    return pl.kernel(
        _sc_body,
        out_type=jax.ShapeDtypeStruct((NC * N_PAD,), jnp.float32),
        mesh=mesh,
        compiler_params=pltpu.CompilerParams(needs_layout_passes=False),
        scratch_types=[
            pltpu.VMEM((N,), jnp.float32),
            pltpu.VMEM((CH, LANE), jnp.int32),
            pltpu.VMEM((CH, LANE), jnp.int32),
            pltpu.VMEM((CH, LANE), jnp.float32),
            pltpu.VMEM((CH, LANE), jnp.int32),
            pltpu.VMEM((CH, LANE), jnp.int32),
            pltpu.VMEM((CH, LANE), jnp.float32),
            pltpu.VMEM_SHARED((N_PAD,), jnp.float32),
            pltpu.SemaphoreType.DMA,
            pltpu.SemaphoreType.DMA,
            pltpu.SemaphoreType.DMA,
            pltpu.SemaphoreType.DMA,
        ],
    )(eidx, w, x)


def _tc_body(x_ref, p_ref, o_ref):
    o_ref[...] = jnp.tanh(x_ref[...] + p_ref[0] + p_ref[1])


def _tc_combine(xp, p):
    return pl.pallas_call(
        _tc_body,
        out_shape=jax.ShapeDtypeStruct((N_PAD // LANE, LANE), jnp.float32),
    )(xp, p)


def kernel(region_inputs_flat, edge_index, edge_weight):
    x = region_inputs_flat
    eidx = edge_index.astype(jnp.int32).reshape(2, ROWS, LANE)
    w = edge_weight.reshape(ROWS, LANE)
    partial = _sc_edges(eidx, w, x)
    xp = jnp.concatenate([x, jnp.zeros((N_PAD - N,), jnp.float32)])
    out2 = _tc_combine(xp.reshape(N_PAD // LANE, LANE),
                       partial.reshape(NC, N_PAD // LANE, LANE))
    return out2.reshape(-1)[:N]


# speedup vs baseline: 1.7870x; 1.0386x over previous
import jax
import jax.numpy as jnp
from jax import lax
from jax.experimental import pallas as pl
from jax.experimental.pallas import tpu as pltpu
from jax.experimental.pallas import tpu_sc as plsc
import functools

N = 100_000
E = 6_400_000
LANE = 128
ROWS = E // LANE
NC, NS = 2, 16
NW = NC * NS
RPW = 1560
XTRA = (ROWS - RPW * NW) // 8
CH = 32
CHE = CH * LANE
NCHUNK = 49
N_PAD = 102_400
SLC = N_PAD // NS


def _sc_body(eidx_hbm, w_hbm, x_hbm, out_hbm,
             x_v, src_v0, dst_v0, w_v0, src_v1, dst_v1, w_v1, acc_sh,
             sem0, sem1, lsem0, lsem1):
    cid = lax.axis_index("c")
    sid = lax.axis_index("s")
    wid = sid * NC + cid

    def _zero(i, c):
        x_v[pl.ds(i * 16, 16)] = jnp.zeros((16,), jnp.float32)
        return c
    lax.fori_loop(0, SLC // 16, _zero, 0)
    off = sid * SLC
    pltpu.sync_copy(x_v.at[pl.ds(0, SLC)], acc_sh.at[pl.ds(off, SLC)])

    pltpu.sync_copy(x_hbm, x_v)
    plsc.subcore_barrier()

    my_rows = RPW + 8 * jnp.where(wid < XTRA, 1, 0)
    r0 = wid * RPW + 8 * jnp.minimum(wid, XTRA)

    def _gather_group(g, src_v, w_v):
        sls = [pl.ds(g * LANE + k * 16, 16) for k in range(LANE // 16)]
        idxs = [src_v[sl] for sl in sls]
        vals = [plsc.load_gather(x_v, [idx]) for idx in idxs]
        for sl, v in zip(sls, vals):
            w_v[sl] = w_v[sl] * v

    def _gather_fire(lo, src_v, dst_v, w_v, sem):
        def _grp(g, cc):
            _gather_group(g, src_v, w_v)
            return cc
        lax.fori_loop(lo, CH, _grp, 0)
        pltpu.async_copy(w_v, acc_sh.at[dst_v], sem, add=True)

    def _issue_loads(ebase, src_v, dst_v, w_v, lsem):
        pltpu.async_copy(eidx_hbm.at[pl.ds(ebase, CHE)], src_v, lsem)
        pltpu.async_copy(eidx_hbm.at[pl.ds(E + ebase, CHE)], dst_v, lsem)
        pltpu.async_copy(w_hbm.at[pl.ds(ebase, CHE)], w_v, lsem)

    def _wait_loads(src_v, dst_v, w_v, lsem):
        pltpu.make_async_copy(eidx_hbm.at[pl.ds(0, CHE)], src_v, lsem).wait()
        pltpu.make_async_copy(eidx_hbm.at[pl.ds(0, CHE)], dst_v, lsem).wait()
        pltpu.make_async_copy(w_hbm.at[pl.ds(0, CHE)], w_v, lsem).wait()

    def _drain(dst_v, w_v, sem):
        pltpu.make_async_copy(w_hbm.at[pl.ds(0, CHE)], w_v, sem).wait()

    set0 = (src_v0, dst_v0, w_v0)
    set1 = (src_v1, dst_v1, w_v1)

    _issue_loads(pl.multiple_of(r0 * LANE, 8), *set0, lsem0)

    def _pair(t, carry):
        _wait_loads(*set0, lsem0)
        _gather_fire(0, *set0, sem0)
        pl.when(t > 0)(lambda: _drain(dst_v1, w_v1, sem1))
        _issue_loads(pl.multiple_of((r0 + (2 * t + 1) * CH) * LANE, 8),
                     *set1, lsem1)
        _wait_loads(*set1, lsem1)
        _gather_fire(0, *set1, sem1)
        _drain(dst_v0, w_v0, sem0)
        nb = jnp.minimum(r0 + (2 * t + 2) * CH, r0 + my_rows - CH)
        _issue_loads(pl.multiple_of(nb * LANE, 8), *set0, lsem0)
        return carry
    lax.fori_loop(0, (NCHUNK - 1) // 2, _pair, 0)

    skip = NCHUNK * CH - my_rows
    _wait_loads(*set0, lsem0)

    def _ztail(i, cc):
        w_v0[pl.ds(i * 16, 16)] = jnp.zeros((16,), jnp.float32)
        return cc
    lax.fori_loop(0, skip * (LANE // 16), _ztail, 0)
    _gather_fire(skip, *set0, sem0)
    _drain(dst_v1, w_v1, sem1)
    _drain(dst_v0, w_v0, sem0)

    plsc.subcore_barrier()
    obase = pl.multiple_of(cid * N_PAD + off, 8)
    pltpu.sync_copy(acc_sh.at[pl.ds(off, SLC)], out_hbm.at[pl.ds(obase, SLC)])


@functools.partial(jax.jit, static_argnames=())
def _sc_edges(eidx, w, x):
    mesh = plsc.VectorSubcoreMesh(core_axis_name="c", subcore_axis_name="s",
                                  num_cores=NC, num_subcores=NS)
    return pl.kernel(
        _sc_body,
        out_type=jax.ShapeDtypeStruct((NC * N_PAD,), jnp.float32),
        mesh=mesh,
        compiler_params=pltpu.CompilerParams(needs_layout_passes=False),
        scratch_types=[
            pltpu.VMEM((N,), jnp.float32),
            pltpu.VMEM((CHE,), jnp.int32),
            pltpu.VMEM((CHE,), jnp.int32),
            pltpu.VMEM((CHE,), jnp.float32),
            pltpu.VMEM((CHE,), jnp.int32),
            pltpu.VMEM((CHE,), jnp.int32),
            pltpu.VMEM((CHE,), jnp.float32),
            pltpu.VMEM_SHARED((N_PAD,), jnp.float32),
            pltpu.SemaphoreType.DMA,
            pltpu.SemaphoreType.DMA,
            pltpu.SemaphoreType.DMA,
            pltpu.SemaphoreType.DMA,
        ],
    )(eidx, w, x)


def _tc_body(x_ref, p_ref, o_ref):
    o_ref[...] = jnp.tanh(x_ref[...] + p_ref[0] + p_ref[1])


def _tc_combine(xp, p):
    return pl.pallas_call(
        _tc_body,
        out_shape=jax.ShapeDtypeStruct((N_PAD // LANE, LANE), jnp.float32),
    )(xp, p)


def kernel(region_inputs_flat, edge_index, edge_weight):
    x = region_inputs_flat
    eflat = edge_index.astype(jnp.int32).reshape(2 * E)
    partial = _sc_edges(eflat, edge_weight, x)
    xp = jnp.concatenate([x, jnp.zeros((N_PAD - N,), jnp.float32)])
    out2 = _tc_combine(xp.reshape(N_PAD // LANE, LANE),
                       partial.reshape(NC, N_PAD // LANE, LANE))
    return out2.reshape(-1)[:N]


# speedup vs baseline: 2.3819x; 1.3329x over previous
import jax
import jax.numpy as jnp
from jax import lax
from jax.experimental import pallas as pl
from jax.experimental.pallas import tpu as pltpu
from jax.experimental.pallas import tpu_sc as plsc
import functools

N = 100_000
E = 6_400_000
LANE = 128
ROWS = E // LANE
NC, NS = 2, 16
NW = NC * NS
RPW = 1560
XTRA = (ROWS - RPW * NW) // 8
CH = 32
CHE = CH * LANE
NCHUNK = 49
N_PAD = 102_400
SLC = N_PAD // NS


def _sc_body(eidx_hbm, w_hbm, x_hbm, out_hbm,
             x_v, src_v0, dst_v0, w_v0, src_v1, dst_v1, w_v1, acc_sh,
             sem0, sem1, lsem0, lsem1):
    cid = lax.axis_index("c")
    sid = lax.axis_index("s")
    wid = sid * NC + cid

    def _zero(i, c):
        x_v[pl.ds(i * 16, 16)] = jnp.zeros((16,), jnp.float32)
        return c
    lax.fori_loop(0, SLC // 16, _zero, 0)
    off = sid * SLC
    pltpu.sync_copy(x_v.at[pl.ds(0, SLC)], acc_sh.at[pl.ds(off, SLC)])

    pltpu.sync_copy(x_hbm, x_v)
    plsc.subcore_barrier()

    my_rows = RPW + 8 * jnp.where(wid < XTRA, 1, 0)
    r0 = wid * RPW + 8 * jnp.minimum(wid, XTRA)

    def _gather_group(g, src_v, w_v):
        sls = [pl.ds(g * LANE + k * 16, 16) for k in range(LANE // 16)]
        idxs = [src_v[sl] for sl in sls]
        vals = [plsc.load_gather(x_v, [idx]) for idx in idxs]
        for sl, v in zip(sls, vals):
            w_v[sl] = w_v[sl] * v

    def _gather_fire(lo, src_v, dst_v, w_v, sem):
        def _grp(g, cc):
            _gather_group(g, src_v, w_v)
            return cc
        lax.fori_loop(lo, CH, _grp, 0)
        pltpu.async_copy(w_v, acc_sh.at[dst_v], sem, add=True)

    def _issue_loads(ebase, src_v, dst_v, w_v, lsem):
        pltpu.async_copy(eidx_hbm.at[0, pl.ds(ebase, CHE)], src_v, lsem)
        pltpu.async_copy(eidx_hbm.at[1, pl.ds(ebase, CHE)], dst_v, lsem)
        pltpu.async_copy(w_hbm.at[pl.ds(ebase, CHE)], w_v, lsem)

    def _wait_loads(src_v, dst_v, w_v, lsem):
        pltpu.make_async_copy(eidx_hbm.at[0, pl.ds(0, CHE)], src_v, lsem).wait()
        pltpu.make_async_copy(eidx_hbm.at[1, pl.ds(0, CHE)], dst_v, lsem).wait()
        pltpu.make_async_copy(w_hbm.at[pl.ds(0, CHE)], w_v, lsem).wait()

    def _drain(dst_v, w_v, sem):
        pltpu.make_async_copy(w_hbm.at[pl.ds(0, CHE)], w_v, sem).wait()

    set0 = (src_v0, dst_v0, w_v0)
    set1 = (src_v1, dst_v1, w_v1)

    _issue_loads(pl.multiple_of(r0 * LANE, 8), *set0, lsem0)

    def _pair(t, carry):
        _wait_loads(*set0, lsem0)
        _gather_fire(0, *set0, sem0)
        pl.when(t > 0)(lambda: _drain(dst_v1, w_v1, sem1))
        _issue_loads(pl.multiple_of((r0 + (2 * t + 1) * CH) * LANE, 8),
                     *set1, lsem1)
        _wait_loads(*set1, lsem1)
        _gather_fire(0, *set1, sem1)
        _drain(dst_v0, w_v0, sem0)
        nb = jnp.minimum(r0 + (2 * t + 2) * CH, r0 + my_rows - CH)
        _issue_loads(pl.multiple_of(nb * LANE, 8), *set0, lsem0)
        return carry
    lax.fori_loop(0, (NCHUNK - 1) // 2, _pair, 0)

    skip = NCHUNK * CH - my_rows
    _wait_loads(*set0, lsem0)

    def _ztail(i, cc):
        w_v0[pl.ds(i * 16, 16)] = jnp.zeros((16,), jnp.float32)
        return cc
    lax.fori_loop(0, skip * (LANE // 16), _ztail, 0)
    _gather_fire(skip, *set0, sem0)
    _drain(dst_v1, w_v1, sem1)
    _drain(dst_v0, w_v0, sem0)

    plsc.subcore_barrier()
    obase = pl.multiple_of(cid * N_PAD + off, 8)
    pltpu.sync_copy(acc_sh.at[pl.ds(off, SLC)], out_hbm.at[pl.ds(obase, SLC)])


@functools.partial(jax.jit, static_argnames=())
def _sc_edges(eidx, w, x):
    mesh = plsc.VectorSubcoreMesh(core_axis_name="c", subcore_axis_name="s",
                                  num_cores=NC, num_subcores=NS)
    return pl.kernel(
        _sc_body,
        out_type=jax.ShapeDtypeStruct((NC * N_PAD,), jnp.float32),
        mesh=mesh,
        compiler_params=pltpu.CompilerParams(needs_layout_passes=False),
        scratch_types=[
            pltpu.VMEM((N,), jnp.float32),
            pltpu.VMEM((CHE,), jnp.int32),
            pltpu.VMEM((CHE,), jnp.int32),
            pltpu.VMEM((CHE,), jnp.float32),
            pltpu.VMEM((CHE,), jnp.int32),
            pltpu.VMEM((CHE,), jnp.int32),
            pltpu.VMEM((CHE,), jnp.float32),
            pltpu.VMEM_SHARED((N_PAD,), jnp.float32),
            pltpu.SemaphoreType.DMA,
            pltpu.SemaphoreType.DMA,
            pltpu.SemaphoreType.DMA,
            pltpu.SemaphoreType.DMA,
        ],
    )(eidx, w, x)


def _tc_body(x_ref, p_ref, o_ref):
    o_ref[...] = jnp.tanh(x_ref[...] + p_ref[0] + p_ref[1])


def _tc_combine(xp, p):
    return pl.pallas_call(
        _tc_body,
        out_shape=jax.ShapeDtypeStruct((N_PAD // LANE, LANE), jnp.float32),
    )(xp, p)


def kernel(region_inputs_flat, edge_index, edge_weight):
    x = region_inputs_flat
    partial = _sc_edges(edge_index.astype(jnp.int32), edge_weight, x)
    xp = jnp.concatenate([x, jnp.zeros((N_PAD - N,), jnp.float32)])
    out2 = _tc_combine(xp.reshape(N_PAD // LANE, LANE),
                       partial.reshape(NC, N_PAD // LANE, LANE))
    return out2.reshape(-1)[:N]


# speedup vs baseline: 2.4111x; 1.0123x over previous
import jax
import jax.numpy as jnp
from jax import lax
from jax.experimental import pallas as pl
from jax.experimental.pallas import tpu as pltpu
from jax.experimental.pallas import tpu_sc as plsc
import functools

N = 100_000
E = 6_400_000
LANE = 128
ROWS = E // LANE
NC, NS = 2, 16
NW = NC * NS
RPW = 1560
XTRA = (ROWS - RPW * NW) // 8
CH = 32
CHE = CH * LANE
NCHUNK = 49
N_PAD = 102_400
SLC = N_PAD // NS


def _sc_body(eidx_hbm, w_hbm, x_hbm, out_hbm,
             x_v, src_v0, dst_v0, w_v0, src_v1, dst_v1, w_v1, acc_sh,
             sem0, sem1, lsem0, lsem1):
    cid = lax.axis_index("c")
    sid = lax.axis_index("s")
    wid = sid * NC + cid

    def _zero(i, c):
        x_v[pl.ds(i * 16, 16)] = jnp.zeros((16,), jnp.float32)
        return c
    lax.fori_loop(0, SLC // 16, _zero, 0)
    off = sid * SLC
    pltpu.sync_copy(x_v.at[pl.ds(0, SLC)], acc_sh.at[pl.ds(off, SLC)])

    pltpu.sync_copy(x_hbm, x_v)
    plsc.subcore_barrier()

    my_rows = RPW + 8 * jnp.where(wid < XTRA, 1, 0)
    r0 = wid * RPW + 8 * jnp.minimum(wid, XTRA)

    def _gather_group(g, src_v, w_v):
        sls = [pl.ds(g * LANE + k * 16, 16) for k in range(LANE // 16)]
        idxs = [src_v[sl] for sl in sls]
        vals = [plsc.load_gather(x_v, [idx]) for idx in idxs]
        for sl, v in zip(sls, vals):
            w_v[sl] = w_v[sl] * v

    def _gather_fire(lo, src_v, dst_v, w_v, sem):
        def _grp(g, cc):
            _gather_group(g, src_v, w_v)
            return cc
        lax.fori_loop(lo, CH, _grp, 0)
        pltpu.async_copy(w_v, acc_sh.at[dst_v], sem, add=True)

    def _issue_loads(ebase, src_v, dst_v, w_v, lsem):
        pltpu.async_copy(eidx_hbm.at[0, pl.ds(ebase, CHE)], src_v, lsem)
        pltpu.async_copy(eidx_hbm.at[1, pl.ds(ebase, CHE)], dst_v, lsem)
        pltpu.async_copy(w_hbm.at[pl.ds(ebase, CHE)], w_v, lsem)

    def _wait_loads(src_v, dst_v, w_v, lsem):
        pltpu.make_async_copy(eidx_hbm.at[0, pl.ds(0, CHE)], src_v, lsem).wait()
        pltpu.make_async_copy(eidx_hbm.at[1, pl.ds(0, CHE)], dst_v, lsem).wait()
        pltpu.make_async_copy(w_hbm.at[pl.ds(0, CHE)], w_v, lsem).wait()

    def _drain(dst_v, w_v, sem):
        pltpu.make_async_copy(w_hbm.at[pl.ds(0, CHE)], w_v, sem).wait()

    set0 = (src_v0, dst_v0, w_v0)
    set1 = (src_v1, dst_v1, w_v1)

    _issue_loads(pl.multiple_of(r0 * LANE, 8), *set0, lsem0)

    def _pair(t, carry):
        _wait_loads(*set0, lsem0)
        _gather_fire(0, *set0, sem0)
        pl.when(t > 0)(lambda: _drain(dst_v1, w_v1, sem1))
        _issue_loads(pl.multiple_of((r0 + (2 * t + 1) * CH) * LANE, 8),
                     *set1, lsem1)
        _wait_loads(*set1, lsem1)
        _gather_fire(0, *set1, sem1)
        _drain(dst_v0, w_v0, sem0)
        nb = jnp.minimum(r0 + (2 * t + 2) * CH, r0 + my_rows - CH)
        _issue_loads(pl.multiple_of(nb * LANE, 8), *set0, lsem0)
        return carry
    lax.fori_loop(0, (NCHUNK - 1) // 2, _pair, 0)

    skip = NCHUNK * CH - my_rows
    _wait_loads(*set0, lsem0)

    def _ztail(i, cc):
        w_v0[pl.ds(i * 16, 16)] = jnp.zeros((16,), jnp.float32)
        return cc
    lax.fori_loop(0, skip * (LANE // 16), _ztail, 0)
    _gather_fire(skip, *set0, sem0)
    _drain(dst_v1, w_v1, sem1)
    _drain(dst_v0, w_v0, sem0)

    plsc.subcore_barrier()
    obase = pl.multiple_of(cid * N_PAD + off, 8)
    pltpu.sync_copy(acc_sh.at[pl.ds(off, SLC)], out_hbm.at[pl.ds(obase, SLC)])


@functools.partial(jax.jit, static_argnames=())
def _sc_edges(eidx, w, x):
    mesh = plsc.VectorSubcoreMesh(core_axis_name="c", subcore_axis_name="s",
                                  num_cores=NC, num_subcores=NS)
    return pl.kernel(
        _sc_body,
        out_type=jax.ShapeDtypeStruct((NC * N_PAD,), jnp.float32),
        mesh=mesh,
        compiler_params=pltpu.CompilerParams(needs_layout_passes=False),
        scratch_types=[
            pltpu.VMEM((N,), jnp.float32),
            pltpu.VMEM((CHE,), jnp.int32),
            pltpu.VMEM((CHE,), jnp.int32),
            pltpu.VMEM((CHE,), jnp.float32),
            pltpu.VMEM((CHE,), jnp.int32),
            pltpu.VMEM((CHE,), jnp.int32),
            pltpu.VMEM((CHE,), jnp.float32),
            pltpu.VMEM_SHARED((N_PAD,), jnp.float32),
            pltpu.SemaphoreType.DMA,
            pltpu.SemaphoreType.DMA,
            pltpu.SemaphoreType.DMA,
            pltpu.SemaphoreType.DMA,
        ],
    )(eidx, w, x)


def _tc_body(x_ref, p_ref, o_ref):
    o_ref[...] = jnp.tanh(x_ref[...] + p_ref[pl.ds(0, N)]
                          + p_ref[pl.ds(N_PAD, N)])


def _tc_combine(x, p):
    return pl.pallas_call(
        _tc_body,
        out_shape=jax.ShapeDtypeStruct((N,), jnp.float32),
    )(x, p)


def kernel(region_inputs_flat, edge_index, edge_weight):
    x = region_inputs_flat
    partial = _sc_edges(edge_index.astype(jnp.int32), edge_weight, x)
    return _tc_combine(x, partial)
